# baseline (speedup 1.0000x reference)
import functools

import jax
import jax.numpy as jnp
from jax import lax
from jax.experimental import pallas as pl
from jax.experimental.pallas import tpu as pltpu
from jax.experimental.pallas import tpu_sc as plsc

NC = 2
NS = 16
NW = NC * NS
L = 16
C = 80
NBUF = 4
NB = 16


def _linear_body(x_ref, w_ref, b_ref, o_ref):
    o_ref[...] = lax.dot_general(
        x_ref[...], w_ref[...], (((1,), (1,)), ((), ())),
        preferred_element_type=jnp.float32) + b_ref[...]


def _combine_body(p_ref, o_ref):
    o_ref[...] = jnp.maximum(p_ref[0] + p_ref[1], 0.0)


def _make_sc_kernel(n_pad, d, e_pad):
    chunks_per_tile = e_pad // (NW * C)
    assert chunks_per_tile % NB == 0
    n_batches = chunks_per_tile // NB
    rows_per_sub = n_pad // NS
    mesh = plsc.VectorSubcoreMesh(
        core_axis_name="c", subcore_axis_name="s",
        num_cores=NC, num_subcores=NS)

    scratch = (
        [pltpu.VMEM((NB * C,), jnp.int32),
         pltpu.VMEM((NB, C), jnp.int32),
         pltpu.VMEM((NB * C,), jnp.float32)]
        + [pltpu.VMEM((C, d), jnp.float32) for _ in range(NBUF)]
        + [pltpu.VMEM_SHARED((n_pad, d), jnp.float32)]
        + [pltpu.SemaphoreType.DMA for _ in range(2 * NBUF)]
    )

    @functools.partial(
        pl.kernel,
        out_type=jax.ShapeDtypeStruct((NC, n_pad, d), jnp.float32),
        mesh=mesh,
        scratch_types=scratch,
    )
    def sc_kernel(h_hbm, src_hbm, dst2_hbm, w_hbm, zeros_hbm, out_hbm, *sc):
        src_a, dst_a, w_a = sc[0:3]
        rows_v = sc[3:3 + NBUF]
        acc_sh = sc[3 + NBUF]
        gsem = sc[4 + NBUF:4 + 2 * NBUF]
        ssem = sc[4 + 2 * NBUF:4 + 3 * NBUF]

        cid = lax.axis_index("c")
        sid = lax.axis_index("s")
        wid = sid * NC + cid
        G = chunks_per_tile
        tile_chunk0 = wid * G

        row0 = sid * rows_per_sub
        pltpu.sync_copy(zeros_hbm.at[pl.ds(row0, rows_per_sub)],
                        acc_sh.at[pl.ds(row0, rows_per_sub)])
        plsc.subcore_barrier()

        def load_batch(nb):
            crow = tile_chunk0 + nb * NB
            pltpu.sync_copy(src_hbm.at[pl.ds(crow * C, NB * C)], src_a)
            pltpu.sync_copy(dst2_hbm.at[pl.ds(crow, NB)], dst_a)
            pltpu.sync_copy(w_hbm.at[pl.ds(crow * C, NB * C)], w_a)

        def start_gather(lc, b):
            pltpu.async_copy(h_hbm.at[src_a.at[pl.ds(lc * C, C)]],
                             rows_v[b], gsem[b])

        def wait_gather(lc, b):
            pltpu.make_async_copy(h_hbm.at[src_a.at[pl.ds(lc * C, C)]],
                                  rows_v[b], gsem[b]).wait()

        def start_scatter(lc, b):
            pltpu.async_copy(rows_v[b], acc_sh.at[dst_a.at[lc]],
                             ssem[b], add=True)

        def wait_scatter(lc, b):
            pltpu.make_async_copy(rows_v[b], acc_sh.at[dst_a.at[lc]],
                                  ssem[b]).wait()

        bcast_idx = [jnp.full((L,), e, jnp.int32) for e in range(L)]

        def scale(lc, b):
            def scale_body(gg, carry):
                w16 = w_a[pl.ds(lc * C + gg * L, L)]
                for e in range(L):
                    wb = jnp.take(w16, bcast_idx[e])
                    r = gg * L + e
                    for jj in range(d // L):
                        s = pl.ds(jj * L, L)
                        rows_v[b][r, s] = rows_v[b][r, s] * wb
                return carry
            lax.fori_loop(0, C // L, scale_body, 0)

        def batch_body(nb, carry):
            load_batch(nb)
            start_gather(0, 0)
            start_gather(1, 1)

            def inner(ii, carry2):
                for j in range(NBUF):
                    lc = ii * NBUF + j
                    bp2 = (j + 2) % NBUF

                    @pl.when(lc >= 2)
                    def _():
                        wait_scatter(lc - 2, bp2)

                    @pl.when(lc + 2 < NB)
                    def _():
                        start_gather(lc + 2, bp2)

                    wait_gather(lc, j)
                    scale(lc, j)
                    start_scatter(lc, j)
                return carry2

            lax.fori_loop(0, NB // NBUF, inner, 0)
            wait_scatter(NB - 2, (NB - 2) % NBUF)
            wait_scatter(NB - 1, (NB - 1) % NBUF)
            return carry

        lax.fori_loop(0, n_batches, batch_body, 0)
        plsc.subcore_barrier()

        pltpu.sync_copy(acc_sh.at[pl.ds(row0, rows_per_sub)],
                        out_hbm.at[cid, pl.ds(row0, rows_per_sub)])

    return sc_kernel


def kernel(feat, edge_index, edge_weight, W, b):
    n, d_in = feat.shape
    d_out = W.shape[0]
    e = edge_index.shape[1]

    src = edge_index[0].astype(jnp.int32)
    dst = edge_index[1].astype(jnp.int32)
    w = edge_weight.reshape(-1).astype(jnp.float32)

    block = NW * C * NB
    e_pad = ((e + block - 1) // block) * block
    if e_pad != e:
        pad = e_pad - e
        src = jnp.concatenate([src, jnp.zeros((pad,), jnp.int32)])
        dst = jnp.concatenate([dst, jnp.zeros((pad,), jnp.int32)])
        w = jnp.concatenate([w, jnp.zeros((pad,), jnp.float32)])
    dst2 = dst.reshape(e_pad // C, C)

    rows_blk = 1000
    grid = n // rows_blk
    h = pl.pallas_call(
        _linear_body,
        grid=(grid,),
        in_specs=[
            pl.BlockSpec((rows_blk, d_in), lambda i: (i, 0)),
            pl.BlockSpec((d_out, d_in), lambda i: (0, 0)),
            pl.BlockSpec((1, d_out), lambda i: (0, 0)),
        ],
        out_specs=pl.BlockSpec((rows_blk, d_out), lambda i: (i, 0)),
        out_shape=jax.ShapeDtypeStruct((n, d_out), jnp.float32),
    )(feat, W, b.reshape(1, d_out))

    n_pad = ((n + 8 * NS - 1) // (8 * NS)) * (8 * NS)
    zeros = jnp.zeros((n_pad, d_out), jnp.float32)
    partials = _make_sc_kernel(n_pad, d_out, e_pad)(h, src, dst2, w, zeros)

    out = pl.pallas_call(
        _combine_body,
        grid=(grid,),
        in_specs=[pl.BlockSpec((NC, rows_blk, d_out), lambda i: (0, i, 0))],
        out_specs=pl.BlockSpec((rows_blk, d_out), lambda i: (i, 0)),
        out_shape=jax.ShapeDtypeStruct((n, d_out), jnp.float32),
    )(partials)
    return out


# speedup vs baseline: 1.0189x; 1.0189x over previous
import functools

import jax
import jax.numpy as jnp
from jax import lax
from jax.experimental import pallas as pl
from jax.experimental.pallas import tpu as pltpu
from jax.experimental.pallas import tpu_sc as plsc

NC = 2
NS = 16
NW = NC * NS
L = 16
C = 80
NBUF = 4
NB = 32


def _linear_body(x_ref, w_ref, b_ref, o_ref):
    o_ref[...] = lax.dot_general(
        x_ref[...], w_ref[...], (((1,), (1,)), ((), ())),
        preferred_element_type=jnp.float32) + b_ref[...]


def _combine_body(p_ref, o_ref):
    o_ref[...] = jnp.maximum(p_ref[0] + p_ref[1], 0.0)


def _make_sc_kernel(n_pad, d, e_pad):
    chunks_per_tile = e_pad // (NW * C)
    assert chunks_per_tile % NB == 0
    n_batches = chunks_per_tile // NB
    rows_per_sub = n_pad // NS
    mesh = plsc.VectorSubcoreMesh(
        core_axis_name="c", subcore_axis_name="s",
        num_cores=NC, num_subcores=NS)

    scratch = (
        [pltpu.VMEM((NB * C,), jnp.int32),
         pltpu.VMEM((NB, C), jnp.int32),
         pltpu.VMEM((NB * C,), jnp.float32)]
        + [pltpu.VMEM((C, d), jnp.float32) for _ in range(NBUF)]
        + [pltpu.VMEM_SHARED((n_pad, d), jnp.float32)]
        + [pltpu.SemaphoreType.DMA for _ in range(2 * NBUF)]
    )

    @functools.partial(
        pl.kernel,
        out_type=jax.ShapeDtypeStruct((NC, n_pad, d), jnp.float32),
        mesh=mesh,
        scratch_types=scratch,
    )
    def sc_kernel(h_hbm, src_hbm, dst2_hbm, w_hbm, zeros_hbm, out_hbm, *sc):
        src_a, dst_a, w_a = sc[0:3]
        rows_v = sc[3:3 + NBUF]
        acc_sh = sc[3 + NBUF]
        gsem = sc[4 + NBUF:4 + 2 * NBUF]
        ssem = sc[4 + 2 * NBUF:4 + 3 * NBUF]

        cid = lax.axis_index("c")
        sid = lax.axis_index("s")
        wid = sid * NC + cid
        G = chunks_per_tile
        tile_chunk0 = wid * G

        row0 = sid * rows_per_sub
        pltpu.sync_copy(zeros_hbm.at[pl.ds(row0, rows_per_sub)],
                        acc_sh.at[pl.ds(row0, rows_per_sub)])
        plsc.subcore_barrier()

        def load_batch(nb):
            crow = tile_chunk0 + nb * NB
            pltpu.sync_copy(src_hbm.at[pl.ds(crow * C, NB * C)], src_a)
            pltpu.sync_copy(dst2_hbm.at[pl.ds(crow, NB)], dst_a)
            pltpu.sync_copy(w_hbm.at[pl.ds(crow * C, NB * C)], w_a)

        def start_gather(lc, b):
            pltpu.async_copy(h_hbm.at[src_a.at[pl.ds(lc * C, C)]],
                             rows_v[b], gsem[b])

        def wait_gather(lc, b):
            pltpu.make_async_copy(h_hbm.at[src_a.at[pl.ds(lc * C, C)]],
                                  rows_v[b], gsem[b]).wait()

        def start_scatter(lc, b):
            pltpu.async_copy(rows_v[b], acc_sh.at[dst_a.at[lc]],
                             ssem[b], add=True)

        def wait_scatter(lc, b):
            pltpu.make_async_copy(rows_v[b], acc_sh.at[dst_a.at[lc]],
                                  ssem[b]).wait()

        bcast_idx = [jnp.full((L,), e, jnp.int32) for e in range(L)]

        def scale(lc, b):
            def scale_body(gg, carry):
                w16 = w_a[pl.ds(lc * C + gg * L, L)]
                for e in range(L):
                    wb = jnp.take(w16, bcast_idx[e])
                    r = gg * L + e
                    for jj in range(d // L):
                        s = pl.ds(jj * L, L)
                        rows_v[b][r, s] = rows_v[b][r, s] * wb
                return carry
            lax.fori_loop(0, C // L, scale_body, 0)

        def batch_body(nb, carry):
            load_batch(nb)
            start_gather(0, 0)
            start_gather(1, 1)

            def inner(ii, carry2):
                for j in range(NBUF):
                    lc = ii * NBUF + j
                    bp2 = (j + 2) % NBUF

                    @pl.when(lc >= 2)
                    def _():
                        wait_scatter(lc - 2, bp2)

                    @pl.when(lc + 2 < NB)
                    def _():
                        start_gather(lc + 2, bp2)

                    wait_gather(lc, j)
                    scale(lc, j)
                    start_scatter(lc, j)
                return carry2

            lax.fori_loop(0, NB // NBUF, inner, 0)
            wait_scatter(NB - 2, (NB - 2) % NBUF)
            wait_scatter(NB - 1, (NB - 1) % NBUF)
            return carry

        lax.fori_loop(0, n_batches, batch_body, 0)
        plsc.subcore_barrier()

        pltpu.sync_copy(acc_sh.at[pl.ds(row0, rows_per_sub)],
                        out_hbm.at[cid, pl.ds(row0, rows_per_sub)])

    return sc_kernel


def kernel(feat, edge_index, edge_weight, W, b):
    n, d_in = feat.shape
    d_out = W.shape[0]
    e = edge_index.shape[1]

    src = edge_index[0].astype(jnp.int32)
    dst = edge_index[1].astype(jnp.int32)
    w = edge_weight.reshape(-1).astype(jnp.float32)

    block = NW * C * NB
    e_pad = ((e + block - 1) // block) * block
    if e_pad != e:
        pad = e_pad - e
        src = jnp.concatenate([src, jnp.zeros((pad,), jnp.int32)])
        dst = jnp.concatenate([dst, jnp.zeros((pad,), jnp.int32)])
        w = jnp.concatenate([w, jnp.zeros((pad,), jnp.float32)])
    dst2 = dst.reshape(e_pad // C, C)

    rows_blk = 1000
    grid = n // rows_blk
    h = pl.pallas_call(
        _linear_body,
        grid=(grid,),
        in_specs=[
            pl.BlockSpec((rows_blk, d_in), lambda i: (i, 0)),
            pl.BlockSpec((d_out, d_in), lambda i: (0, 0)),
            pl.BlockSpec((1, d_out), lambda i: (0, 0)),
        ],
        out_specs=pl.BlockSpec((rows_blk, d_out), lambda i: (i, 0)),
        out_shape=jax.ShapeDtypeStruct((n, d_out), jnp.float32),
    )(feat, W, b.reshape(1, d_out))

    n_pad = ((n + 8 * NS - 1) // (8 * NS)) * (8 * NS)
    zeros = jnp.zeros((n_pad, d_out), jnp.float32)
    partials = _make_sc_kernel(n_pad, d_out, e_pad)(h, src, dst2, w, zeros)

    out = pl.pallas_call(
        _combine_body,
        grid=(grid,),
        in_specs=[pl.BlockSpec((NC, rows_blk, d_out), lambda i: (0, i, 0))],
        out_specs=pl.BlockSpec((rows_blk, d_out), lambda i: (i, 0)),
        out_shape=jax.ShapeDtypeStruct((n, d_out), jnp.float32),
    )(partials)
    return out
